# baseline (speedup 1.0000x reference)
import functools

import jax
import jax.numpy as jnp
from jax import lax
from jax.experimental import pallas as pl
from jax.experimental.pallas import tpu as pltpu
from jax.experimental.pallas import tpu_sc as plsc

_NC = 2
_NS = 16
_NW = _NC * _NS
_D = 64
_C = 128


@functools.partial(jax.jit, static_argnums=(2, 3))
def _sc_gather(table, idx3, n_chunks, c):
    n_rows = _NW * n_chunks * c
    mesh = plsc.VectorSubcoreMesh(core_axis_name="c", subcore_axis_name="s")

    @functools.partial(
        pl.kernel,
        out_type=jax.ShapeDtypeStruct((n_rows, _D), jnp.float32),
        mesh=mesh,
        scratch_types=[
            pltpu.VMEM((n_chunks, c), jnp.int32),
            pltpu.VMEM((2, c, _D), jnp.float32),
            pltpu.SemaphoreType.DMA,
            pltpu.SemaphoreType.DMA,
        ],
        compiler_params=pltpu.CompilerParams(use_tc_tiling_on_sc=False),
    )
    def gather_kernel(table_hbm, idx_hbm, out_hbm, idx_v, rows_v, sem0, sem1):
        wid = lax.axis_index("s") * _NC + lax.axis_index("c")
        base = wid * (n_chunks * c)
        pltpu.sync_copy(idx_hbm.at[wid], idx_v)
        sems = (sem0, sem1)
        pltpu.async_copy(table_hbm.at[idx_v.at[0]], rows_v.at[0], sem0)
        pltpu.async_copy(table_hbm.at[idx_v.at[1]], rows_v.at[1], sem1)

        def body(jj, carry):
            j0 = jj * 2
            for slot in range(2):
                j = j0 + slot
                pltpu.make_async_copy(
                    table_hbm.at[idx_v.at[j]], rows_v.at[slot], sems[slot]
                ).wait()
                pltpu.sync_copy(rows_v.at[slot], out_hbm.at[pl.ds(base + j * c, c)])

                @pl.when(j + 2 < n_chunks)
                def _():
                    pltpu.async_copy(
                        table_hbm.at[idx_v.at[j + 2]], rows_v.at[slot], sems[slot]
                    )

            return carry

        lax.fori_loop(0, n_chunks // 2, body, 0)

    return gather_kernel(table, idx3)


def _mm_body(x_ref, wt_ref, b_ref, o_ref):
    o_ref[...] = (
        jnp.dot(x_ref[...], wt_ref[...], preferred_element_type=jnp.float32)
        + b_ref[...]
    )


@functools.partial(jax.jit, static_argnums=(3,))
def _tc_linear(x, wt, b2, bm):
    n_rows = x.shape[0]
    return pl.pallas_call(
        _mm_body,
        grid=(n_rows // bm,),
        in_specs=[
            pl.BlockSpec((bm, _D), lambda i: (i, 0)),
            pl.BlockSpec((_D, _D), lambda i: (0, 0)),
            pl.BlockSpec((1, _D), lambda i: (0, 0)),
        ],
        out_specs=pl.BlockSpec((bm, _D), lambda i: (i, 0)),
        out_shape=jax.ShapeDtypeStruct((n_rows, _D), jnp.float32),
    )(x, wt, b2)


def kernel(entity, table, W, b):
    batch, hist = entity.shape
    n_rows = batch * hist
    n_chunks = n_rows // (_NW * _C)
    idx3 = entity.reshape(_NW, n_chunks, _C)
    gathered = _sc_gather(table, idx3, n_chunks, _C)
    out = _tc_linear(gathered, W.T, b.reshape(1, _D), 8192)
    return out.reshape(batch, hist, _D)


# speedup vs baseline: 1.0247x; 1.0247x over previous
import functools

import jax
import jax.numpy as jnp
from jax import lax
from jax.experimental import pallas as pl
from jax.experimental.pallas import tpu as pltpu
from jax.experimental.pallas import tpu_sc as plsc

_NC = 2
_NS = 16
_NW = _NC * _NS
_D = 64
_C = 128
_BK = 4096


def _proj_body(x_ref, w_ref, b_ref, o_ref):
    y = jax.lax.dot_general(
        x_ref[...], w_ref[...], (((0,), (1,)), ((), ())),
        preferred_element_type=jnp.float32,
    )
    o_ref[...] = y + b_ref[...]


@jax.jit
def _project_table(table_t, w, b2):
    n = table_t.shape[1]
    return pl.pallas_call(
        _proj_body,
        grid=(pl.cdiv(n, _BK),),
        in_specs=[
            pl.BlockSpec((_D, _BK), lambda i: (0, i)),
            pl.BlockSpec((_D, _D), lambda i: (0, 0)),
            pl.BlockSpec((1, _D), lambda i: (0, 0)),
        ],
        out_specs=pl.BlockSpec((_BK, _D), lambda i: (i, 0)),
        out_shape=jax.ShapeDtypeStruct((n, _D), jnp.float32),
    )(table_t, w, b2)


@functools.partial(jax.jit, static_argnums=(2, 3))
def _sc_gather(ptable, idx3, n_chunks, c):
    n_rows = _NW * n_chunks * c
    mesh = plsc.VectorSubcoreMesh(core_axis_name="c", subcore_axis_name="s")

    @functools.partial(
        pl.kernel,
        out_type=jax.ShapeDtypeStruct((n_rows, _D), jnp.float32),
        mesh=mesh,
        scratch_types=[
            pltpu.VMEM((n_chunks, c), jnp.int32),
            pltpu.VMEM((2, c, _D), jnp.float32),
            pltpu.SemaphoreType.DMA,
            pltpu.SemaphoreType.DMA,
        ],
        compiler_params=pltpu.CompilerParams(use_tc_tiling_on_sc=False),
    )
    def gather_kernel(table_hbm, idx_hbm, out_hbm, idx_v, rows_v, sem0, sem1):
        wid = lax.axis_index("s") * _NC + lax.axis_index("c")
        base = wid * (n_chunks * c)
        pltpu.sync_copy(idx_hbm.at[wid], idx_v)
        sems = (sem0, sem1)
        pltpu.async_copy(table_hbm.at[idx_v.at[0]], rows_v.at[0], sem0)
        pltpu.async_copy(table_hbm.at[idx_v.at[1]], rows_v.at[1], sem1)

        def body(jj, carry):
            j0 = jj * 2
            for slot in range(2):
                j = j0 + slot
                pltpu.make_async_copy(
                    table_hbm.at[idx_v.at[j]], rows_v.at[slot], sems[slot]
                ).wait()
                pltpu.sync_copy(rows_v.at[slot], out_hbm.at[pl.ds(base + j * c, c)])

                @pl.when(j + 2 < n_chunks)
                def _():
                    pltpu.async_copy(
                        table_hbm.at[idx_v.at[j + 2]], rows_v.at[slot], sems[slot]
                    )

            return carry

        lax.fori_loop(0, n_chunks // 2, body, 0)

    return gather_kernel(ptable, idx3)


def _tr_body(g_ref, i_ref, o_ref):
    o_ref[0] = jax.lax.dot_general(
        i_ref[...], g_ref[0], (((1,), (1,)), ((), ())),
        preferred_element_type=jnp.float32,
    )


@jax.jit
def _transpose_out(g3, ident):
    hist, batch, _ = g3.shape
    return pl.pallas_call(
        _tr_body,
        grid=(hist,),
        in_specs=[
            pl.BlockSpec((1, batch, _D), lambda j: (j, 0, 0)),
            pl.BlockSpec((_D, _D), lambda j: (0, 0)),
        ],
        out_specs=pl.BlockSpec((1, _D, batch), lambda j: (j, 0, 0)),
        out_shape=jax.ShapeDtypeStruct((hist, _D, batch), jnp.float32),
    )(g3, ident)


def kernel(entity, table, W, b):
    batch, hist = entity.shape
    n_rows = batch * hist
    n_chunks = n_rows // (_NW * _C)

    ptable = _project_table(table.T, W, b.reshape(1, _D))

    idx3 = entity.T.reshape(_NW, n_chunks, _C)
    g = _sc_gather(ptable, idx3, n_chunks, _C)

    ident = jnp.eye(_D, dtype=jnp.float32)
    ot = _transpose_out(g.reshape(hist, batch, _D), ident)

    return jnp.transpose(ot, (2, 0, 1))


# speedup vs baseline: 1.5333x; 1.4963x over previous
import functools

import jax
import jax.numpy as jnp
from jax import lax
from jax.experimental import pallas as pl
from jax.experimental.pallas import tpu as pltpu
from jax.experimental.pallas import tpu_sc as plsc

_NC = 2
_NS = 16
_NW = _NC * _NS
_D = 64
_C = 128
_BK = 4096


def _proj_body(x_ref, w_ref, b_ref, o_ref):
    y = jax.lax.dot_general(
        x_ref[...], w_ref[...], (((0,), (1,)), ((), ())),
        preferred_element_type=jnp.float32,
    )
    o_ref[:, : _D] = y + b_ref[...]


@jax.jit
def _project_table(table_t, w, b2):
    n = table_t.shape[1]
    return pl.pallas_call(
        _proj_body,
        grid=(pl.cdiv(n, _BK),),
        in_specs=[
            pl.BlockSpec((_D, _BK), lambda i: (0, i)),
            pl.BlockSpec((_D, _D), lambda i: (0, 0)),
            pl.BlockSpec((1, _D), lambda i: (0, 0)),
        ],
        out_specs=pl.BlockSpec((_BK, 2 * _D), lambda i: (i, 0)),
        out_shape=jax.ShapeDtypeStruct((n, 2 * _D), jnp.float32),
    )(table_t, w, b2)


@functools.partial(jax.jit, static_argnums=(2, 3))
def _sc_gather(ptable, idx3, n_chunks, c):
    n_rows = _NW * n_chunks * c
    mesh = plsc.VectorSubcoreMesh(core_axis_name="c", subcore_axis_name="s")

    @functools.partial(
        pl.kernel,
        out_type=jax.ShapeDtypeStruct((n_rows, _D), jnp.float32),
        mesh=mesh,
        scratch_types=[
            pltpu.VMEM((n_chunks, c), jnp.int32),
            pltpu.VMEM((2, c, _D), jnp.float32),
            pltpu.SemaphoreType.DMA,
            pltpu.SemaphoreType.DMA,
        ],
        compiler_params=pltpu.CompilerParams(use_tc_tiling_on_sc=False),
    )
    def gather_kernel(table_hbm, idx_hbm, out_hbm, idx_v, rows_v, sem0, sem1):
        wid = lax.axis_index("s") * _NC + lax.axis_index("c")
        base = wid * (n_chunks * c)
        pltpu.sync_copy(idx_hbm.at[wid], idx_v)
        sems = (sem0, sem1)
        pltpu.async_copy(table_hbm.at[idx_v.at[0]], rows_v.at[0], sem0)
        pltpu.async_copy(table_hbm.at[idx_v.at[1]], rows_v.at[1], sem1)

        def body(jj, carry):
            j0 = jj * 2
            for slot in range(2):
                j = j0 + slot
                pltpu.make_async_copy(
                    table_hbm.at[idx_v.at[j]], rows_v.at[slot], sems[slot]
                ).wait()
                pltpu.sync_copy(rows_v.at[slot], out_hbm.at[pl.ds(base + j * c, c)])

                @pl.when(j + 2 < n_chunks)
                def _():
                    pltpu.async_copy(
                        table_hbm.at[idx_v.at[j + 2]], rows_v.at[slot], sems[slot]
                    )

            return carry

        lax.fori_loop(0, n_chunks // 2, body, 0)

    return gather_kernel(ptable, idx3)


def _tr_body(g_ref, i_ref, o_ref):
    half = g_ref.shape[1]
    o_ref[0, :, :half] = jax.lax.dot_general(
        i_ref[...], g_ref[0, :, : _D], (((1,), (1,)), ((), ())),
        preferred_element_type=jnp.float32,
    )
    o_ref[0, :, half:] = jax.lax.dot_general(
        i_ref[...], g_ref[0, :, _D :], (((1,), (1,)), ((), ())),
        preferred_element_type=jnp.float32,
    )


@jax.jit
def _transpose_out(gp, ident):
    hist, half, _ = gp.shape
    return pl.pallas_call(
        _tr_body,
        grid=(hist,),
        in_specs=[
            pl.BlockSpec((1, half, 2 * _D), lambda j: (j, 0, 0)),
            pl.BlockSpec((_D, _D), lambda j: (0, 0)),
        ],
        out_specs=pl.BlockSpec((1, _D, 2 * half), lambda j: (j, 0, 0)),
        out_shape=jax.ShapeDtypeStruct((hist, _D, 2 * half), jnp.float32),
    )(gp, ident)


def kernel(entity, table, W, b):
    batch, hist = entity.shape
    n_rows = batch * hist
    n_chunks = n_rows // (_NW * _C)
    half = batch // 2

    ptable = _project_table(table.T, W, b.reshape(1, _D))
    ptable_rows = ptable.reshape(2 * table.shape[0], _D)

    et = entity.T.reshape(hist, 2, half)
    idx_perm = jnp.transpose(et, (0, 2, 1))
    idx3 = (idx_perm * 2).reshape(_NW, n_chunks, _C)
    g = _sc_gather(ptable_rows, idx3, n_chunks, _C)

    ident = jnp.eye(_D, dtype=jnp.float32)
    gp = g.reshape(hist, half, 2 * _D)
    ot = _transpose_out(gp, ident)

    return jnp.transpose(ot, (2, 0, 1))


# speedup vs baseline: 2.0062x; 1.3084x over previous
import functools

import jax
import jax.numpy as jnp
from jax import lax
from jax.experimental import pallas as pl
from jax.experimental.pallas import tpu as pltpu
from jax.experimental.pallas import tpu_sc as plsc

_NC = 2
_NS = 16
_NW = _NC * _NS
_D = 64
_C = 128
_BK = 4096


def _proj_body(x_ref, w_ref, b_ref, o_ref):
    y = jax.lax.dot_general(
        x_ref[...], w_ref[...], (((0,), (1,)), ((), ())),
        preferred_element_type=jnp.float32,
    )
    o_ref[:, : _D] = y + b_ref[...]


@jax.jit
def _project_table(table_t, w, b2):
    n = table_t.shape[1]
    return pl.pallas_call(
        _proj_body,
        grid=(pl.cdiv(n, _BK),),
        in_specs=[
            pl.BlockSpec((_D, _BK), lambda i: (0, i)),
            pl.BlockSpec((_D, _D), lambda i: (0, 0)),
            pl.BlockSpec((1, _D), lambda i: (0, 0)),
        ],
        out_specs=pl.BlockSpec((_BK, 2 * _D), lambda i: (i, 0)),
        out_shape=jax.ShapeDtypeStruct((n, 2 * _D), jnp.float32),
    )(table_t, w, b2)


@functools.partial(jax.jit, static_argnums=(2,))
def _sc_gather(ptable, idx3, n_units):
    hist, n_k2, c = idx3.shape
    n_k = n_k2 // 2
    n_out = hist * n_k * c
    mesh = plsc.VectorSubcoreMesh(core_axis_name="c", subcore_axis_name="s")

    @functools.partial(
        pl.kernel,
        out_type=jax.ShapeDtypeStruct((n_out, 2 * _D), jnp.float32),
        mesh=mesh,
        scratch_types=[
            pltpu.VMEM((n_units, c), jnp.int32),
            pltpu.VMEM((n_units, c), jnp.int32),
            pltpu.VMEM((2, c, _D), jnp.float32),
            pltpu.VMEM((2, c, _D), jnp.float32),
            pltpu.SemaphoreType.DMA,
            pltpu.SemaphoreType.DMA,
            pltpu.SemaphoreType.DMA,
            pltpu.SemaphoreType.DMA,
        ],
        compiler_params=pltpu.CompilerParams(use_tc_tiling_on_sc=False),
    )
    def gather_kernel(table_hbm, idx_hbm, out_hbm, idx_lo, idx_hi,
                      stage_lo, stage_hi, gs0, gs1, ws0, ws1):
        wid = lax.axis_index("s") * _NC + lax.axis_index("c")
        kk = wid % n_k
        j0 = (wid // n_k) * n_units
        pltpu.sync_copy(idx_hbm.at[pl.ds(j0, n_units), kk], idx_lo)
        pltpu.sync_copy(idx_hbm.at[pl.ds(j0, n_units), n_k + kk], idx_hi)
        gsems = (gs0, gs1)
        wsems = (ws0, ws1)

        def row0(m):
            return (j0 + m) * (n_k * c) + kk * c

        pltpu.async_copy(table_hbm.at[idx_lo.at[0]], stage_lo.at[0], gs0)
        pltpu.async_copy(table_hbm.at[idx_hi.at[0]], stage_hi.at[0], gs0)

        def pair_body(p, carry):
            for h in range(2):
                m = p * 2 + h
                pltpu.make_async_copy(
                    table_hbm.at[idx_lo.at[m]], stage_lo.at[h], gsems[h]).wait()
                pltpu.make_async_copy(
                    table_hbm.at[idx_hi.at[m]], stage_hi.at[h], gsems[h]).wait()
                pltpu.async_copy(
                    stage_lo.at[h],
                    out_hbm.at[pl.ds(row0(m), c), pl.ds(0, _D)], wsems[h])
                pltpu.async_copy(
                    stage_hi.at[h],
                    out_hbm.at[pl.ds(row0(m), c), pl.ds(_D, _D)], wsems[h])

                @pl.when(m >= 1)
                def _():
                    pltpu.make_async_copy(
                        stage_lo.at[1 - h],
                        out_hbm.at[pl.ds(row0(m - 1), c), pl.ds(0, _D)],
                        wsems[1 - h]).wait()
                    pltpu.make_async_copy(
                        stage_hi.at[1 - h],
                        out_hbm.at[pl.ds(row0(m - 1), c), pl.ds(_D, _D)],
                        wsems[1 - h]).wait()

                @pl.when(m + 1 < n_units)
                def _():
                    pltpu.async_copy(
                        table_hbm.at[idx_lo.at[m + 1]], stage_lo.at[1 - h],
                        gsems[1 - h])
                    pltpu.async_copy(
                        table_hbm.at[idx_hi.at[m + 1]], stage_hi.at[1 - h],
                        gsems[1 - h])

            return carry

        lax.fori_loop(0, n_units // 2, pair_body, 0)
        pltpu.make_async_copy(
            stage_lo.at[1],
            out_hbm.at[pl.ds(row0(n_units - 1), c), pl.ds(0, _D)], wsems[1]).wait()
        pltpu.make_async_copy(
            stage_hi.at[1],
            out_hbm.at[pl.ds(row0(n_units - 1), c), pl.ds(_D, _D)], wsems[1]).wait()

    return gather_kernel(ptable, idx3)


def _tr_body(g_ref, i_ref, o_ref):
    half = g_ref.shape[1]
    o_ref[0, :, :half] = jax.lax.dot_general(
        i_ref[...], g_ref[0, :, : _D], (((1,), (1,)), ((), ())),
        preferred_element_type=jnp.float32,
    )
    o_ref[0, :, half:] = jax.lax.dot_general(
        i_ref[...], g_ref[0, :, _D :], (((1,), (1,)), ((), ())),
        preferred_element_type=jnp.float32,
    )


@jax.jit
def _transpose_out(gp, ident):
    hist, half, _ = gp.shape
    return pl.pallas_call(
        _tr_body,
        grid=(hist,),
        in_specs=[
            pl.BlockSpec((1, half, 2 * _D), lambda j: (j, 0, 0)),
            pl.BlockSpec((_D, _D), lambda j: (0, 0)),
        ],
        out_specs=pl.BlockSpec((1, _D, 2 * half), lambda j: (j, 0, 0)),
        out_shape=jax.ShapeDtypeStruct((hist, _D, 2 * half), jnp.float32),
    )(gp, ident)


def kernel(entity, table, W, b):
    batch, hist = entity.shape
    half = batch // 2
    n_k = batch // (2 * _C)

    ptable = _project_table(table.T, W, b.reshape(1, _D))
    ptable_rows = ptable.reshape(2 * table.shape[0], _D)

    idx3 = (entity.T * 2).reshape(hist, 2 * n_k, _C)
    n_units = (hist * n_k) // _NW
    g = _sc_gather(ptable_rows, idx3, n_units)

    ident = jnp.eye(_D, dtype=jnp.float32)
    gp = g.reshape(hist, half, 2 * _D)
    ot = _transpose_out(gp, ident)

    return jnp.transpose(ot, (2, 0, 1))


# speedup vs baseline: 2.4225x; 1.2075x over previous
import functools

import jax
import jax.numpy as jnp
from jax import lax
from jax.experimental import pallas as pl
from jax.experimental.pallas import tpu as pltpu
from jax.experimental.pallas import tpu_sc as plsc

_NC = 2
_NS = 16
_NW = _NC * _NS
_D = 64
_C = 128
_BK = 8192


def _proj_body(x_ref, w_ref, b_ref, o_ref):
    y = jax.lax.dot_general(
        x_ref[...], w_ref[...], (((0,), (1,)), ((), ())),
        preferred_element_type=jnp.float32,
    )
    o_ref[:, : _D] = y + b_ref[...]


@jax.jit
def _project_table(table_t, w, b2):
    n = table_t.shape[1]
    return pl.pallas_call(
        _proj_body,
        grid=(pl.cdiv(n, _BK),),
        in_specs=[
            pl.BlockSpec((_D, _BK), lambda i: (0, i)),
            pl.BlockSpec((_D, _D), lambda i: (0, 0)),
            pl.BlockSpec((1, _D), lambda i: (0, 0)),
        ],
        out_specs=pl.BlockSpec((_BK, 2 * _D), lambda i: (i, 0)),
        out_shape=jax.ShapeDtypeStruct((n, 2 * _D), jnp.float32),
    )(table_t, w, b2)


@functools.partial(jax.jit, static_argnums=(2,))
def _sc_gather(ptable, idx3, n_units):
    hist, n_k2, c = idx3.shape
    n_k = n_k2 // 2
    n_out = hist * n_k * c
    mesh = plsc.VectorSubcoreMesh(core_axis_name="c", subcore_axis_name="s")

    @functools.partial(
        pl.kernel,
        out_type=jax.ShapeDtypeStruct((n_out, 2 * _D), jnp.float32),
        mesh=mesh,
        scratch_types=[
            pltpu.VMEM((n_units, c), jnp.int32),
            pltpu.VMEM((n_units, c), jnp.int32),
            pltpu.VMEM((2, c, _D), jnp.float32),
            pltpu.VMEM((2, c, _D), jnp.float32),
            pltpu.SemaphoreType.DMA,
            pltpu.SemaphoreType.DMA,
            pltpu.SemaphoreType.DMA,
            pltpu.SemaphoreType.DMA,
        ],
        compiler_params=pltpu.CompilerParams(use_tc_tiling_on_sc=False),
    )
    def gather_kernel(table_hbm, idx_hbm, out_hbm, idx_lo, idx_hi,
                      stage_lo, stage_hi, gs0, gs1, ws0, ws1):
        wid = lax.axis_index("s") * _NC + lax.axis_index("c")
        kk = wid % n_k
        j0 = (wid // n_k) * n_units
        pltpu.sync_copy(idx_hbm.at[pl.ds(j0, n_units), kk], idx_lo)
        pltpu.sync_copy(idx_hbm.at[pl.ds(j0, n_units), n_k + kk], idx_hi)
        gsems = (gs0, gs1)
        wsems = (ws0, ws1)

        def row0(m):
            return (j0 + m) * (n_k * c) + kk * c

        pltpu.async_copy(table_hbm.at[idx_lo.at[0]], stage_lo.at[0], gs0)
        pltpu.async_copy(table_hbm.at[idx_hi.at[0]], stage_hi.at[0], gs0)

        def pair_body(p, carry):
            for h in range(2):
                m = p * 2 + h
                pltpu.make_async_copy(
                    table_hbm.at[idx_lo.at[m]], stage_lo.at[h], gsems[h]).wait()
                pltpu.make_async_copy(
                    table_hbm.at[idx_hi.at[m]], stage_hi.at[h], gsems[h]).wait()
                pltpu.async_copy(
                    stage_lo.at[h],
                    out_hbm.at[pl.ds(row0(m), c), pl.ds(0, _D)], wsems[h])
                pltpu.async_copy(
                    stage_hi.at[h],
                    out_hbm.at[pl.ds(row0(m), c), pl.ds(_D, _D)], wsems[h])

                @pl.when(m >= 1)
                def _():
                    pltpu.make_async_copy(
                        stage_lo.at[1 - h],
                        out_hbm.at[pl.ds(row0(m - 1), c), pl.ds(0, _D)],
                        wsems[1 - h]).wait()
                    pltpu.make_async_copy(
                        stage_hi.at[1 - h],
                        out_hbm.at[pl.ds(row0(m - 1), c), pl.ds(_D, _D)],
                        wsems[1 - h]).wait()

                @pl.when(m + 1 < n_units)
                def _():
                    pltpu.async_copy(
                        table_hbm.at[idx_lo.at[m + 1]], stage_lo.at[1 - h],
                        gsems[1 - h])
                    pltpu.async_copy(
                        table_hbm.at[idx_hi.at[m + 1]], stage_hi.at[1 - h],
                        gsems[1 - h])

            return carry

        lax.fori_loop(0, n_units // 2, pair_body, 0)
        pltpu.make_async_copy(
            stage_lo.at[1],
            out_hbm.at[pl.ds(row0(n_units - 1), c), pl.ds(0, _D)], wsems[1]).wait()
        pltpu.make_async_copy(
            stage_hi.at[1],
            out_hbm.at[pl.ds(row0(n_units - 1), c), pl.ds(_D, _D)], wsems[1]).wait()

    return gather_kernel(ptable, idx3)


def _tr_body(g_ref, i_ref, o_ref):
    half = g_ref.shape[1]
    for jj in range(2):
        o_ref[jj, :, :half] = jax.lax.dot_general(
            i_ref[...], g_ref[jj, :, : _D], (((1,), (1,)), ((), ())),
            preferred_element_type=jnp.float32,
        )
        o_ref[jj, :, half:] = jax.lax.dot_general(
            i_ref[...], g_ref[jj, :, _D :], (((1,), (1,)), ((), ())),
            preferred_element_type=jnp.float32,
        )


@jax.jit
def _transpose_out(gp, ident):
    hist, half, _ = gp.shape
    return pl.pallas_call(
        _tr_body,
        grid=(hist // 2,),
        in_specs=[
            pl.BlockSpec((2, half, 2 * _D), lambda j: (j, 0, 0)),
            pl.BlockSpec((_D, _D), lambda j: (0, 0)),
        ],
        out_specs=pl.BlockSpec((2, _D, 2 * half), lambda j: (j, 0, 0)),
        out_shape=jax.ShapeDtypeStruct((hist, _D, 2 * half), jnp.float32),
    )(gp, ident)


def kernel(entity, table, W, b):
    batch, hist = entity.shape
    half = batch // 2
    n_k = batch // (2 * _C)

    ptable = _project_table(table.T, W, b.reshape(1, _D))
    ptable_rows = ptable.reshape(2 * table.shape[0], _D)

    idx3 = (entity.T * 2).reshape(hist, 2 * n_k, _C)
    n_units = (hist * n_k) // _NW
    g = _sc_gather(ptable_rows, idx3, n_units)

    ident = jnp.eye(_D, dtype=jnp.float32)
    gp = g.reshape(hist, half, 2 * _D)
    ot = _transpose_out(gp, ident)

    return jnp.transpose(ot, (2, 0, 1))


# speedup vs baseline: 2.5328x; 1.0455x over previous
import functools

import jax
import jax.numpy as jnp
from jax import lax
from jax.experimental import pallas as pl
from jax.experimental.pallas import tpu as pltpu
from jax.experimental.pallas import tpu_sc as plsc

_NC = 2
_NS = 16
_NW = _NC * _NS
_D = 64
_C = 128
_BK = 8192


def _proj_body(x_ref, w_ref, b_ref, o_ref):
    y = jax.lax.dot_general(
        x_ref[...], w_ref[...], (((0,), (1,)), ((), ())),
        preferred_element_type=jnp.float32,
    )
    o_ref[:, : _D] = y + b_ref[...]


@jax.jit
def _project_table(table_t, w, b2):
    n = table_t.shape[1]
    return pl.pallas_call(
        _proj_body,
        grid=(pl.cdiv(n, _BK),),
        in_specs=[
            pl.BlockSpec((_D, _BK), lambda i: (0, i)),
            pl.BlockSpec((_D, _D), lambda i: (0, 0)),
            pl.BlockSpec((1, _D), lambda i: (0, 0)),
        ],
        out_specs=pl.BlockSpec((_BK, 2 * _D), lambda i: (i, 0)),
        out_shape=jax.ShapeDtypeStruct((n, 2 * _D), jnp.float32),
    )(table_t, w, b2)


@functools.partial(jax.jit, static_argnums=(2,))
def _sc_gather(ptable, idx3, n_units):
    hist, n_k2, c = idx3.shape
    n_k = n_k2 // 2
    n_out = hist * n_k * c
    mesh = plsc.VectorSubcoreMesh(core_axis_name="c", subcore_axis_name="s")

    @functools.partial(
        pl.kernel,
        out_type=jax.ShapeDtypeStruct((n_out, 2 * _D), jnp.float32),
        mesh=mesh,
        scratch_types=[
            pltpu.VMEM((n_units, c), jnp.int32),
            pltpu.VMEM((n_units, c), jnp.int32),
            pltpu.VMEM((2, c, _D), jnp.float32),
            pltpu.VMEM((2, c, _D), jnp.float32),
            pltpu.SemaphoreType.DMA,
            pltpu.SemaphoreType.DMA,
            pltpu.SemaphoreType.DMA,
            pltpu.SemaphoreType.DMA,
        ],
        compiler_params=pltpu.CompilerParams(use_tc_tiling_on_sc=False),
    )
    def gather_kernel(table_hbm, idx_hbm, out_hbm, idx_lo, idx_hi,
                      stage_lo, stage_hi, gs0, gs1, ws0, ws1):
        wid = lax.axis_index("s") * _NC + lax.axis_index("c")
        kk = wid % n_k
        j0 = (wid // n_k) * n_units
        pltpu.sync_copy(idx_hbm.at[pl.ds(j0, n_units), kk], idx_lo)
        pltpu.sync_copy(idx_hbm.at[pl.ds(j0, n_units), n_k + kk], idx_hi)
        gsems = (gs0, gs1)
        wsems = (ws0, ws1)

        def row0(m):
            return (j0 + m) * (n_k * c) + kk * c

        pltpu.async_copy(table_hbm.at[idx_lo.at[0]], stage_lo.at[0], gs0)
        pltpu.async_copy(table_hbm.at[idx_hi.at[0]], stage_hi.at[0], gs0)

        def pair_body(p, carry):
            for h in range(2):
                m = p * 2 + h
                pltpu.make_async_copy(
                    table_hbm.at[idx_lo.at[m]], stage_lo.at[h], gsems[h]).wait()
                pltpu.make_async_copy(
                    table_hbm.at[idx_hi.at[m]], stage_hi.at[h], gsems[h]).wait()
                pltpu.async_copy(
                    stage_lo.at[h],
                    out_hbm.at[pl.ds(row0(m), c), pl.ds(0, _D)], wsems[h])
                pltpu.async_copy(
                    stage_hi.at[h],
                    out_hbm.at[pl.ds(row0(m), c), pl.ds(_D, _D)], wsems[h])

                @pl.when(m >= 1)
                def _():
                    pltpu.make_async_copy(
                        stage_lo.at[1 - h],
                        out_hbm.at[pl.ds(row0(m - 1), c), pl.ds(0, _D)],
                        wsems[1 - h]).wait()
                    pltpu.make_async_copy(
                        stage_hi.at[1 - h],
                        out_hbm.at[pl.ds(row0(m - 1), c), pl.ds(_D, _D)],
                        wsems[1 - h]).wait()

                @pl.when(m + 1 < n_units)
                def _():
                    pltpu.async_copy(
                        table_hbm.at[idx_lo.at[m + 1]], stage_lo.at[1 - h],
                        gsems[1 - h])
                    pltpu.async_copy(
                        table_hbm.at[idx_hi.at[m + 1]], stage_hi.at[1 - h],
                        gsems[1 - h])

            return carry

        lax.fori_loop(0, n_units // 2, pair_body, 0)
        pltpu.make_async_copy(
            stage_lo.at[1],
            out_hbm.at[pl.ds(row0(n_units - 1), c), pl.ds(0, _D)], wsems[1]).wait()
        pltpu.make_async_copy(
            stage_hi.at[1],
            out_hbm.at[pl.ds(row0(n_units - 1), c), pl.ds(_D, _D)], wsems[1]).wait()

    return gather_kernel(ptable, idx3)


def _tr_body(g_ref, i_ref, o_ref):
    half = g_ref.shape[1]
    for jj in range(2):
        o_ref[jj, :, :half] = jax.lax.dot_general(
            i_ref[...], g_ref[jj, :, : _D], (((1,), (1,)), ((), ())),
            preferred_element_type=jnp.float32,
        )
        o_ref[jj, :, half:] = jax.lax.dot_general(
            i_ref[...], g_ref[jj, :, _D :], (((1,), (1,)), ((), ())),
            preferred_element_type=jnp.float32,
        )


@functools.partial(jax.jit, static_argnums=(2, 3))
def _transpose_out_first(gp, ident, hist_total, j_off):
    _, half, _ = gp.shape
    hist = gp.shape[0]
    return pl.pallas_call(
        _tr_body,
        grid=(hist // 2,),
        in_specs=[
            pl.BlockSpec((2, half, 2 * _D), lambda j: (j, 0, 0)),
            pl.BlockSpec((_D, _D), lambda j: (0, 0)),
        ],
        out_specs=pl.BlockSpec((2, _D, 2 * half), lambda j: (j + j_off // 2, 0, 0)),
        out_shape=jax.ShapeDtypeStruct((hist_total, _D, 2 * half), jnp.float32),
    )(gp, ident)


@functools.partial(jax.jit, static_argnums=(3, 4))
def _transpose_out_into(gp, ident, acc, hist_total, j_off):
    _, half, _ = gp.shape
    hist = gp.shape[0]
    return pl.pallas_call(
        _tr_body2,
        grid=(hist // 2,),
        in_specs=[
            pl.BlockSpec((2, half, 2 * _D), lambda j: (j, 0, 0)),
            pl.BlockSpec((_D, _D), lambda j: (0, 0)),
            pl.BlockSpec(memory_space=pl.ANY),
        ],
        out_specs=pl.BlockSpec((2, _D, 2 * half), lambda j: (j + j_off // 2, 0, 0)),
        out_shape=jax.ShapeDtypeStruct((hist_total, _D, 2 * half), jnp.float32),
        input_output_aliases={2: 0},
    )(gp, ident, acc)


def _tr_body2(g_ref, i_ref, acc_ref, o_ref):
    _tr_body(g_ref, i_ref, o_ref)


def kernel(entity, table, W, b):
    batch, hist = entity.shape
    half = batch // 2
    n_k = batch // (2 * _C)
    hh = hist // 2

    ptable = _project_table(table.T, W, b.reshape(1, _D))
    ptable_rows = ptable.reshape(2 * table.shape[0], _D)

    idx3 = (entity.T * 2).reshape(hist, 2 * n_k, _C)
    n_units = (hh * n_k) // _NW
    g1 = _sc_gather(ptable_rows, idx3[:hh], n_units)
    g2 = _sc_gather(ptable_rows, idx3[hh:], n_units)

    ident = jnp.eye(_D, dtype=jnp.float32)
    ot1 = _transpose_out_first(g1.reshape(hh, half, 2 * _D), ident, hist, 0)
    ot = _transpose_out_into(g2.reshape(hh, half, 2 * _D), ident, ot1, hist, hh)

    return jnp.transpose(ot, (2, 0, 1))


# speedup vs baseline: 2.6940x; 1.0636x over previous
import functools

import jax
import jax.numpy as jnp
from jax import lax
from jax.experimental import pallas as pl
from jax.experimental.pallas import tpu as pltpu
from jax.experimental.pallas import tpu_sc as plsc

_NC = 2
_NS = 16
_NW = _NC * _NS
_D = 64
_C = 128
_BK = 16384


def _proj_body(x_ref, w_ref, b_ref, o_ref):
    y = jax.lax.dot_general(
        x_ref[...], w_ref[...], (((0,), (1,)), ((), ())),
        preferred_element_type=jnp.float32,
    )
    o_ref[:, : _D] = y + b_ref[...]


@jax.jit
def _project_table(table_t, w, b2):
    n = table_t.shape[1]
    return pl.pallas_call(
        _proj_body,
        grid=(pl.cdiv(n, _BK),),
        in_specs=[
            pl.BlockSpec((_D, _BK), lambda i: (0, i)),
            pl.BlockSpec((_D, _D), lambda i: (0, 0)),
            pl.BlockSpec((1, _D), lambda i: (0, 0)),
        ],
        out_specs=pl.BlockSpec((_BK, 2 * _D), lambda i: (i, 0)),
        out_shape=jax.ShapeDtypeStruct((n, 2 * _D), jnp.float32),
    )(table_t, w, b2)


@functools.partial(jax.jit, static_argnums=(2,))
def _sc_gather(ptable, idx3, n_units):
    hist, n_k2, c = idx3.shape
    n_k = n_k2 // 2
    n_out = hist * n_k * c
    mesh = plsc.VectorSubcoreMesh(core_axis_name="c", subcore_axis_name="s")

    @functools.partial(
        pl.kernel,
        out_type=jax.ShapeDtypeStruct((n_out, 2 * _D), jnp.float32),
        mesh=mesh,
        scratch_types=[
            pltpu.VMEM((n_units, c), jnp.int32),
            pltpu.VMEM((n_units, c), jnp.int32),
            pltpu.VMEM((2, c, _D), jnp.float32),
            pltpu.VMEM((2, c, _D), jnp.float32),
            pltpu.SemaphoreType.DMA,
            pltpu.SemaphoreType.DMA,
            pltpu.SemaphoreType.DMA,
            pltpu.SemaphoreType.DMA,
        ],
        compiler_params=pltpu.CompilerParams(use_tc_tiling_on_sc=False),
    )
    def gather_kernel(table_hbm, idx_hbm, out_hbm, idx_lo, idx_hi,
                      stage_lo, stage_hi, gs0, gs1, ws0, ws1):
        wid = lax.axis_index("s") * _NC + lax.axis_index("c")
        kk = wid % n_k
        j0 = (wid // n_k) * n_units
        pltpu.sync_copy(idx_hbm.at[pl.ds(j0, n_units), kk], idx_lo)
        pltpu.sync_copy(idx_hbm.at[pl.ds(j0, n_units), n_k + kk], idx_hi)
        gsems = (gs0, gs1)
        wsems = (ws0, ws1)

        def row0(m):
            return (j0 + m) * (n_k * c) + kk * c

        pltpu.async_copy(table_hbm.at[idx_lo.at[0]], stage_lo.at[0], gs0)
        pltpu.async_copy(table_hbm.at[idx_hi.at[0]], stage_hi.at[0], gs0)

        def pair_body(p, carry):
            for h in range(2):
                m = p * 2 + h
                pltpu.make_async_copy(
                    table_hbm.at[idx_lo.at[m]], stage_lo.at[h], gsems[h]).wait()
                pltpu.make_async_copy(
                    table_hbm.at[idx_hi.at[m]], stage_hi.at[h], gsems[h]).wait()
                pltpu.async_copy(
                    stage_lo.at[h],
                    out_hbm.at[pl.ds(row0(m), c), pl.ds(0, _D)], wsems[h])
                pltpu.async_copy(
                    stage_hi.at[h],
                    out_hbm.at[pl.ds(row0(m), c), pl.ds(_D, _D)], wsems[h])

                @pl.when(m >= 1)
                def _():
                    pltpu.make_async_copy(
                        stage_lo.at[1 - h],
                        out_hbm.at[pl.ds(row0(m - 1), c), pl.ds(0, _D)],
                        wsems[1 - h]).wait()
                    pltpu.make_async_copy(
                        stage_hi.at[1 - h],
                        out_hbm.at[pl.ds(row0(m - 1), c), pl.ds(_D, _D)],
                        wsems[1 - h]).wait()

                @pl.when(m + 1 < n_units)
                def _():
                    pltpu.async_copy(
                        table_hbm.at[idx_lo.at[m + 1]], stage_lo.at[1 - h],
                        gsems[1 - h])
                    pltpu.async_copy(
                        table_hbm.at[idx_hi.at[m + 1]], stage_hi.at[1 - h],
                        gsems[1 - h])

            return carry

        lax.fori_loop(0, n_units // 2, pair_body, 0)
        pltpu.make_async_copy(
            stage_lo.at[1],
            out_hbm.at[pl.ds(row0(n_units - 1), c), pl.ds(0, _D)], wsems[1]).wait()
        pltpu.make_async_copy(
            stage_hi.at[1],
            out_hbm.at[pl.ds(row0(n_units - 1), c), pl.ds(_D, _D)], wsems[1]).wait()

    return gather_kernel(ptable, idx3)


def _tr_body(g_ref, i_ref, o_ref):
    half = g_ref.shape[1]
    for jj in range(g_ref.shape[0]):
        o_ref[jj, :, :half] = jax.lax.dot_general(
            i_ref[...], g_ref[jj, :, : _D], (((1,), (1,)), ((), ())),
            preferred_element_type=jnp.float32,
        )
        o_ref[jj, :, half:] = jax.lax.dot_general(
            i_ref[...], g_ref[jj, :, _D :], (((1,), (1,)), ((), ())),
            preferred_element_type=jnp.float32,
        )


@functools.partial(jax.jit, static_argnums=(2, 3))
def _transpose_out_first(gp, ident, hist_total, j_off):
    _, half, _ = gp.shape
    hist = gp.shape[0]
    return pl.pallas_call(
        _tr_body,
        grid=(hist // 4,),
        in_specs=[
            pl.BlockSpec((4, half, 2 * _D), lambda j: (j, 0, 0)),
            pl.BlockSpec((_D, _D), lambda j: (0, 0)),
        ],
        out_specs=pl.BlockSpec((4, _D, 2 * half), lambda j: (j + j_off // 4, 0, 0)),
        out_shape=jax.ShapeDtypeStruct((hist_total, _D, 2 * half), jnp.float32),
    )(gp, ident)


@functools.partial(jax.jit, static_argnums=(3, 4))
def _transpose_out_into(gp, ident, acc, hist_total, j_off):
    _, half, _ = gp.shape
    hist = gp.shape[0]
    return pl.pallas_call(
        _tr_body2,
        grid=(hist // 4,),
        in_specs=[
            pl.BlockSpec((4, half, 2 * _D), lambda j: (j, 0, 0)),
            pl.BlockSpec((_D, _D), lambda j: (0, 0)),
            pl.BlockSpec(memory_space=pl.ANY),
        ],
        out_specs=pl.BlockSpec((4, _D, 2 * half), lambda j: (j + j_off // 4, 0, 0)),
        out_shape=jax.ShapeDtypeStruct((hist_total, _D, 2 * half), jnp.float32),
        input_output_aliases={2: 0},
    )(gp, ident, acc)


def _tr_body2(g_ref, i_ref, acc_ref, o_ref):
    _tr_body(g_ref, i_ref, o_ref)


def kernel(entity, table, W, b):
    batch, hist = entity.shape
    half = batch // 2
    n_k = batch // (2 * _C)
    hh = hist // 2

    ptable = _project_table(table.T, W, b.reshape(1, _D))
    ptable_rows = ptable.reshape(2 * table.shape[0], _D)

    idx3 = (entity.T * 2).reshape(hist, 2 * n_k, _C)
    n_units = (hh * n_k) // _NW
    g1 = _sc_gather(ptable_rows, idx3[:hh], n_units)
    g2 = _sc_gather(ptable_rows, idx3[hh:], n_units)

    ident = jnp.eye(_D, dtype=jnp.float32)
    ot1 = _transpose_out_first(g1.reshape(hh, half, 2 * _D), ident, hist, 0)
    ot = _transpose_out_into(g2.reshape(hh, half, 2 * _D), ident, ot1, hist, hh)

    return jnp.transpose(ot, (2, 0, 1))


# speedup vs baseline: 2.7816x; 1.0325x over previous
import functools

import jax
import jax.numpy as jnp
from jax import lax
from jax.experimental import pallas as pl
from jax.experimental.pallas import tpu as pltpu
from jax.experimental.pallas import tpu_sc as plsc

_NC = 2
_NS = 16
_NW = _NC * _NS
_D = 64
_C = 128
_BK = 32768


def _proj_body(x_ref, w_ref, b_ref, o_ref):
    y = jax.lax.dot_general(
        x_ref[...], w_ref[...], (((0,), (1,)), ((), ())),
        preferred_element_type=jnp.float32,
    )
    o_ref[:, : _D] = y + b_ref[...]


@jax.jit
def _project_table(table_t, w, b2):
    n = table_t.shape[1]
    return pl.pallas_call(
        _proj_body,
        grid=(pl.cdiv(n, _BK),),
        in_specs=[
            pl.BlockSpec((_D, _BK), lambda i: (0, i)),
            pl.BlockSpec((_D, _D), lambda i: (0, 0)),
            pl.BlockSpec((1, _D), lambda i: (0, 0)),
        ],
        out_specs=pl.BlockSpec((_BK, 2 * _D), lambda i: (i, 0)),
        out_shape=jax.ShapeDtypeStruct((n, 2 * _D), jnp.float32),
    )(table_t, w, b2)


@functools.partial(jax.jit, static_argnums=(2,))
def _sc_gather(ptable, idx3, n_units):
    hist, n_k2, c = idx3.shape
    n_k = n_k2 // 2
    n_out = hist * n_k * c
    mesh = plsc.VectorSubcoreMesh(core_axis_name="c", subcore_axis_name="s")

    @functools.partial(
        pl.kernel,
        out_type=jax.ShapeDtypeStruct((n_out, 2 * _D), jnp.float32),
        mesh=mesh,
        scratch_types=[
            pltpu.VMEM((n_units, c), jnp.int32),
            pltpu.VMEM((n_units, c), jnp.int32),
            pltpu.VMEM((2, c, _D), jnp.float32),
            pltpu.VMEM((2, c, _D), jnp.float32),
            pltpu.SemaphoreType.DMA,
            pltpu.SemaphoreType.DMA,
            pltpu.SemaphoreType.DMA,
            pltpu.SemaphoreType.DMA,
        ],
        compiler_params=pltpu.CompilerParams(use_tc_tiling_on_sc=False),
    )
    def gather_kernel(table_hbm, idx_hbm, out_hbm, idx_lo, idx_hi,
                      stage_lo, stage_hi, gs0, gs1, ws0, ws1):
        wid = lax.axis_index("s") * _NC + lax.axis_index("c")
        kk = wid % n_k
        j0 = (wid // n_k) * n_units
        pltpu.sync_copy(idx_hbm.at[pl.ds(j0, n_units), kk], idx_lo)
        pltpu.sync_copy(idx_hbm.at[pl.ds(j0, n_units), n_k + kk], idx_hi)
        gsems = (gs0, gs1)
        wsems = (ws0, ws1)

        def row0(m):
            return (j0 + m) * (n_k * c) + kk * c

        pltpu.async_copy(table_hbm.at[idx_lo.at[0]], stage_lo.at[0], gs0)
        pltpu.async_copy(table_hbm.at[idx_hi.at[0]], stage_hi.at[0], gs0)

        def pair_body(p, carry):
            for h in range(2):
                m = p * 2 + h
                pltpu.make_async_copy(
                    table_hbm.at[idx_lo.at[m]], stage_lo.at[h], gsems[h]).wait()
                pltpu.make_async_copy(
                    table_hbm.at[idx_hi.at[m]], stage_hi.at[h], gsems[h]).wait()
                pltpu.async_copy(
                    stage_lo.at[h],
                    out_hbm.at[pl.ds(row0(m), c), pl.ds(0, _D)], wsems[h])
                pltpu.async_copy(
                    stage_hi.at[h],
                    out_hbm.at[pl.ds(row0(m), c), pl.ds(_D, _D)], wsems[h])

                @pl.when(m >= 1)
                def _():
                    pltpu.make_async_copy(
                        stage_lo.at[1 - h],
                        out_hbm.at[pl.ds(row0(m - 1), c), pl.ds(0, _D)],
                        wsems[1 - h]).wait()
                    pltpu.make_async_copy(
                        stage_hi.at[1 - h],
                        out_hbm.at[pl.ds(row0(m - 1), c), pl.ds(_D, _D)],
                        wsems[1 - h]).wait()

                @pl.when(m + 1 < n_units)
                def _():
                    pltpu.async_copy(
                        table_hbm.at[idx_lo.at[m + 1]], stage_lo.at[1 - h],
                        gsems[1 - h])
                    pltpu.async_copy(
                        table_hbm.at[idx_hi.at[m + 1]], stage_hi.at[1 - h],
                        gsems[1 - h])

            return carry

        lax.fori_loop(0, n_units // 2, pair_body, 0)
        pltpu.make_async_copy(
            stage_lo.at[1],
            out_hbm.at[pl.ds(row0(n_units - 1), c), pl.ds(0, _D)], wsems[1]).wait()
        pltpu.make_async_copy(
            stage_hi.at[1],
            out_hbm.at[pl.ds(row0(n_units - 1), c), pl.ds(_D, _D)], wsems[1]).wait()

    return gather_kernel(ptable, idx3)


def _tr_body(g_ref, i_ref, o_ref):
    half = g_ref.shape[1]
    for jj in range(g_ref.shape[0]):
        o_ref[jj, :, :half] = jax.lax.dot_general(
            i_ref[...], g_ref[jj, :, : _D], (((1,), (1,)), ((), ())),
            preferred_element_type=jnp.float32,
        )
        o_ref[jj, :, half:] = jax.lax.dot_general(
            i_ref[...], g_ref[jj, :, _D :], (((1,), (1,)), ((), ())),
            preferred_element_type=jnp.float32,
        )


@functools.partial(jax.jit, static_argnums=(2, 3))
def _transpose_out_first(gp, ident, hist_total, j_off):
    _, half, _ = gp.shape
    hist = gp.shape[0]
    return pl.pallas_call(
        _tr_body,
        grid=(hist // 8,),
        in_specs=[
            pl.BlockSpec((8, half, 2 * _D), lambda j: (j, 0, 0)),
            pl.BlockSpec((_D, _D), lambda j: (0, 0)),
        ],
        out_specs=pl.BlockSpec((8, _D, 2 * half), lambda j: (j + j_off // 8, 0, 0)),
        out_shape=jax.ShapeDtypeStruct((hist_total, _D, 2 * half), jnp.float32),
    )(gp, ident)


@functools.partial(jax.jit, static_argnums=(3, 4))
def _transpose_out_into(gp, ident, acc, hist_total, j_off):
    _, half, _ = gp.shape
    hist = gp.shape[0]
    return pl.pallas_call(
        _tr_body2,
        grid=(hist // 8,),
        in_specs=[
            pl.BlockSpec((8, half, 2 * _D), lambda j: (j, 0, 0)),
            pl.BlockSpec((_D, _D), lambda j: (0, 0)),
            pl.BlockSpec(memory_space=pl.ANY),
        ],
        out_specs=pl.BlockSpec((8, _D, 2 * half), lambda j: (j + j_off // 8, 0, 0)),
        out_shape=jax.ShapeDtypeStruct((hist_total, _D, 2 * half), jnp.float32),
        input_output_aliases={2: 0},
    )(gp, ident, acc)


def _tr_body2(g_ref, i_ref, acc_ref, o_ref):
    _tr_body(g_ref, i_ref, o_ref)


def kernel(entity, table, W, b):
    batch, hist = entity.shape
    half = batch // 2
    n_k = batch // (2 * _C)
    hh = hist // 2

    ptable = _project_table(table.T, W, b.reshape(1, _D))
    ptable_rows = ptable.reshape(2 * table.shape[0], _D)

    idx3 = (entity.T * 2).reshape(hist, 2 * n_k, _C)
    n_units = (hh * n_k) // _NW
    g1 = _sc_gather(ptable_rows, idx3[:hh], n_units)
    g2 = _sc_gather(ptable_rows, idx3[hh:], n_units)

    ident = jnp.eye(_D, dtype=jnp.float32)
    ot1 = _transpose_out_first(g1.reshape(hh, half, 2 * _D), ident, hist, 0)
    ot = _transpose_out_into(g2.reshape(hh, half, 2 * _D), ident, ot1, hist, hh)

    return jnp.transpose(ot, (2, 0, 1))
